# baseline (speedup 1.0000x reference)
import functools

import jax
import jax.numpy as jnp
from jax import lax
from jax.experimental import pallas as pl
from jax.experimental.pallas import tpu as pltpu
from jax.experimental.pallas import tpu_sc as plsc

C = 64
SIDE = 1024
HW = SIDE * SIDE
NC, NS, L = 2, 16, 16
NW = NC * NS
B = 128
NBUF = 2


_RB = 8


def _tr_body(ga_ref, gb_ref, t_ref):
    a = ga_ref[...].reshape(C, _RB * SIDE)
    b = gb_ref[...].reshape(C, _RB * SIDE)
    t_ref[:, 0:C] = a.T
    t_ref[:, C:2 * C] = b.T


_transpose = pl.pallas_call(
    _tr_body,
    grid=(SIDE // 2 // _RB,),
    in_specs=[
        pl.BlockSpec((C, _RB, SIDE), lambda i: (0, i, 0)),
        pl.BlockSpec((C, _RB, SIDE), lambda i: (0, i + SIDE // 2 // _RB, 0)),
    ],
    out_specs=pl.BlockSpec((_RB * SIDE, 2 * C), lambda i: (i, 0)),
    out_shape=jax.ShapeDtypeStruct((HW // 2, 2 * C), jnp.float32),
)


def _sc_body(nc0, nc1, coords_hbm, table_hbm, out_hbm, *sc):
    xv = sc[0:2]
    yv = sc[2:4]
    idx = (sc[4:8], sc[8:12])
    wgt = (sc[12:16], sc[16:20])
    taps = (sc[20:24], sc[24:28])
    out_v = sc[28:30]
    gsem = (sc[30:34], sc[34:38])
    osem = sc[38:40]

    cid = lax.axis_index("c")
    sid = lax.axis_index("s")
    nchunk = jnp.where(cid == 0, nc0, nc1)
    wbase = jnp.where(cid == 0, sid * (nc0 * B),
                      NS * nc0 * B + sid * (nc1 * B))

    def stage_a(g, slot):
        base = wbase + g * B
        pltpu.sync_copy(coords_hbm.at[0, pl.ds(base, B)], xv[slot])
        pltpu.sync_copy(coords_hbm.at[1, pl.ds(base, B)], yv[slot])
        for j in range(B // L):
            sj = pl.ds(j * L, L)
            xc = xv[slot][sj]
            yc = yv[slot][sj]
            fx = ((xc + 1.0) * jnp.float32(SIDE) - 1.0) * 0.5
            fy = ((yc + 1.0) * jnp.float32(SIDE) - 1.0) * 0.5
            x0 = fx.astype(jnp.int32)
            y0 = fy.astype(jnp.int32)
            wx1 = fx - x0.astype(jnp.float32)
            wy1 = fy - y0.astype(jnp.float32)
            wx0 = 1.0 - wx1
            wy0 = 1.0 - wy1
            x1 = x0 + 1
            y1 = y0 + 1
            wx1 = jnp.where(x1 <= SIDE - 1, wx1, 0.0)
            wy1 = jnp.where(y1 <= SIDE - 1, wy1, 0.0)
            x1c = jnp.minimum(x1, SIDE - 1)
            y1c = jnp.minimum(y1, SIDE - 1)
            r0 = y0 * SIDE
            r1 = y1c * SIDE
            sl = pl.ds(j * L, L)

            def rmap(p):
                return 2 * (p & (HW // 2 - 1)) + (p >> 19)

            idx[slot][0][sl] = rmap(r0 + x0)
            idx[slot][1][sl] = rmap(r0 + x1c)
            idx[slot][2][sl] = rmap(r1 + x0)
            idx[slot][3][sl] = rmap(r1 + x1c)
            wgt[slot][0][sl] = wx0 * wy0
            wgt[slot][1][sl] = wx1 * wy0
            wgt[slot][2][sl] = wx0 * wy1
            wgt[slot][3][sl] = wx1 * wy1
        for t in range(4):
            pltpu.async_copy(table_hbm.at[idx[slot][t]], taps[slot][t],
                             gsem[slot][t])

    def stage_b(g, slot):
        base = wbase + g * B
        for t in range(4):
            pltpu.make_async_copy(table_hbm.at[idx[slot][t]], taps[slot][t],
                                  gsem[slot][t]).wait()

        @pl.when(g >= NBUF)
        def _():
            prev = wbase + (g - NBUF) * B
            pltpu.make_async_copy(
                out_v[slot], out_hbm.at[pl.ds(prev, B)], osem[slot]).wait()

        def blend_body(jg, _):
            sl = pl.ds(jg * L, L)
            wv = [wgt[slot][t][sl] for t in range(4)]
            for lane in range(L):
                i = jg * L + lane
                a = [jnp.full((L,), wv[t][lane], jnp.float32)
                     for t in range(4)]
                for k in range(C // L):
                    sk = pl.ds(k * L, L)
                    acc = (taps[slot][0][i, sk] * a[0]
                           + taps[slot][1][i, sk] * a[1]
                           + taps[slot][2][i, sk] * a[2]
                           + taps[slot][3][i, sk] * a[3])
                    out_v[slot][i, sk] = acc
            return ()

        lax.fori_loop(0, B // L, blend_body, ())
        pltpu.async_copy(out_v[slot], out_hbm.at[pl.ds(base, B)], osem[slot])

    stage_a(jnp.int32(0), 0)

    def pair_body(gp, _):
        for b in range(NBUF):
            g = gp * NBUF + b
            nxt = g + 1

            @pl.when(nxt < nchunk)
            def _():
                stage_a(nxt, (b + 1) % NBUF)

            stage_b(g, b)
        return ()

    lax.fori_loop(0, nchunk // NBUF, pair_body, ())

    for b in range(NBUF):
        g = nchunk - NBUF + b
        base = wbase + g * B
        pltpu.make_async_copy(
            out_v[b], out_hbm.at[pl.ds(base, B)], osem[b]).wait()


_CORE0_FRAC = 0.573


def _make_sc_sample(npad):
    ntot = npad // (NS * B)
    nc0 = max(2, (int(ntot * _CORE0_FRAC) // 2) * 2)
    nc1 = ntot - nc0
    mesh = plsc.VectorSubcoreMesh(
        core_axis_name="c", subcore_axis_name="s",
        num_cores=NC, num_subcores=NS)
    scratch = []
    scratch += [pltpu.VMEM((B,), jnp.float32) for _ in range(2 * NBUF)]
    scratch += [pltpu.VMEM((B,), jnp.int32) for _ in range(4 * NBUF)]
    scratch += [pltpu.VMEM((B,), jnp.float32) for _ in range(4 * NBUF)]
    scratch += [pltpu.VMEM((B, C), jnp.float32) for _ in range(4 * NBUF)]
    scratch += [pltpu.VMEM((B, 128), jnp.float32) for _ in range(NBUF)]
    scratch += [pltpu.SemaphoreType.DMA for _ in range(5 * NBUF)]
    return pl.kernel(
        functools.partial(_sc_body, nc0, nc1),
        out_type=jax.ShapeDtypeStruct((npad, 128), jnp.float32),
        mesh=mesh,
        compiler_params=pltpu.CompilerParams(
            use_tc_tiling_on_sc=False, needs_layout_passes=False),
        scratch_types=scratch,
    )


def kernel(coords, grid):
    n = coords.shape[0]
    step = NS * B * NBUF
    npad = ((n + step - 1) // step) * step
    table = _transpose(grid, grid).reshape(HW, C)
    ct = coords.T
    if npad != n:
        ct = jnp.pad(ct, ((0, 0), (0, npad - n)))
    out = _make_sc_sample(npad)(ct, table)
    return out[:n, :C], lax.optimization_barrier(coords)


# speedup vs baseline: 1.0978x; 1.0978x over previous
import functools

import jax
import jax.numpy as jnp
from jax import lax
from jax.experimental import pallas as pl
from jax.experimental.pallas import tpu as pltpu
from jax.experimental.pallas import tpu_sc as plsc

C = 64
SIDE = 1024
HW = SIDE * SIDE
NC, NS, L = 2, 16, 16
NW = NC * NS
B = 128
NBUF = 2


_RB = 8
_Q = SIDE // 4 // _RB


def _pack_half(t):
    lo = lax.bitcast_convert_type(t[:, 0:C // 2].astype(jnp.bfloat16),
                                  jnp.uint16).astype(jnp.uint32)
    hi = lax.bitcast_convert_type(t[:, C // 2:C].astype(jnp.bfloat16),
                                  jnp.uint16).astype(jnp.uint32)
    return lo | (hi << 16)


def _tr_body(g0, g1, g2, g3, t_ref):
    for j, g in enumerate((g0, g1, g2, g3)):
        t = g[...].reshape(C, _RB * SIDE).T
        t_ref[:, j * (C // 2):(j + 1) * (C // 2)] = _pack_half(t)


_transpose = pl.pallas_call(
    _tr_body,
    grid=(_Q,),
    in_specs=[
        pl.BlockSpec((C, _RB, SIDE), lambda i, j=j: (0, i + j * _Q, 0))
        for j in range(4)
    ],
    out_specs=pl.BlockSpec((_RB * SIDE, 2 * C), lambda i: (i, 0)),
    out_shape=jax.ShapeDtypeStruct((HW // 4, 2 * C), jnp.uint32),
)


def _sc_body(nc0, nc1, coords_hbm, table_hbm, out_hbm, *sc):
    xv = sc[0:2]
    yv = sc[2:4]
    idx = (sc[4:8], sc[8:12])
    wgt = (sc[12:16], sc[16:20])
    taps = (sc[20:24], sc[24:28])
    out_v = sc[28:30]
    gsem = (sc[30:34], sc[34:38])
    osem = sc[38:40]

    cid = lax.axis_index("c")
    sid = lax.axis_index("s")
    nchunk = jnp.where(cid == 0, nc0, nc1)
    wbase = jnp.where(cid == 0, sid * (nc0 * B),
                      NS * nc0 * B + sid * (nc1 * B))

    def stage_a(g, slot):
        base = wbase + g * B
        pltpu.sync_copy(coords_hbm.at[0, pl.ds(base, B)], xv[slot])
        pltpu.sync_copy(coords_hbm.at[1, pl.ds(base, B)], yv[slot])
        for j in range(B // L):
            sj = pl.ds(j * L, L)
            xc = xv[slot][sj]
            yc = yv[slot][sj]
            fx = ((xc + 1.0) * jnp.float32(SIDE) - 1.0) * 0.5
            fy = ((yc + 1.0) * jnp.float32(SIDE) - 1.0) * 0.5
            x0 = fx.astype(jnp.int32)
            y0 = fy.astype(jnp.int32)
            wx1 = fx - x0.astype(jnp.float32)
            wy1 = fy - y0.astype(jnp.float32)
            wx0 = 1.0 - wx1
            wy0 = 1.0 - wy1
            x1 = x0 + 1
            y1 = y0 + 1
            wx1 = jnp.where(x1 <= SIDE - 1, wx1, 0.0)
            wy1 = jnp.where(y1 <= SIDE - 1, wy1, 0.0)
            x1c = jnp.minimum(x1, SIDE - 1)
            y1c = jnp.minimum(y1, SIDE - 1)
            r0 = y0 * SIDE
            r1 = y1c * SIDE
            sl = pl.ds(j * L, L)

            def rmap(p):
                return 4 * (p & (HW // 4 - 1)) + (p >> 18)

            idx[slot][0][sl] = rmap(r0 + x0)
            idx[slot][1][sl] = rmap(r0 + x1c)
            idx[slot][2][sl] = rmap(r1 + x0)
            idx[slot][3][sl] = rmap(r1 + x1c)
            wgt[slot][0][sl] = wx0 * wy0
            wgt[slot][1][sl] = wx1 * wy0
            wgt[slot][2][sl] = wx0 * wy1
            wgt[slot][3][sl] = wx1 * wy1
        for t in range(4):
            pltpu.async_copy(table_hbm.at[idx[slot][t]], taps[slot][t],
                             gsem[slot][t])

    def stage_b(g, slot):
        base = wbase + g * B
        for t in range(4):
            pltpu.make_async_copy(table_hbm.at[idx[slot][t]], taps[slot][t],
                                  gsem[slot][t]).wait()

        @pl.when(g >= NBUF)
        def _():
            prev = wbase + (g - NBUF) * B
            pltpu.make_async_copy(
                out_v[slot],
                out_hbm.at[pl.ds(prev * C // 128, B * C // 128)],
                osem[slot]).wait()

        def blend_body(jg, _):
            sl = pl.ds(jg * L, L)
            wv = [wgt[slot][t][sl] for t in range(4)]
            for lane in range(L):
                i = jg * L + lane
                a = [jnp.full((L,), wv[t][lane], jnp.float32)
                     for t in range(4)]
                orow = jg * (L // 2) + lane // 2
                cb = (lane % 2) * C
                for k in range(2):
                    tp = [plsc.unpack(
                        plsc.bitcast(taps[slot][t][i, pl.ds(k * L, L)],
                                     jnp.bfloat16),
                        format=plsc.PackFormat.INTERLEAVED)
                        for t in range(4)]
                    lo = (tp[0][0] * a[0] + tp[1][0] * a[1]
                          + tp[2][0] * a[2] + tp[3][0] * a[3])
                    hi = (tp[0][1] * a[0] + tp[1][1] * a[1]
                          + tp[2][1] * a[2] + tp[3][1] * a[3])
                    out_v[slot][orow, pl.ds(cb + k * L, L)] = lo
                    out_v[slot][orow, pl.ds(cb + 2 * L + k * L, L)] = hi
            return ()

        lax.fori_loop(0, B // L, blend_body, ())
        pltpu.async_copy(out_v[slot],
                         out_hbm.at[pl.ds(base * C // 128, B * C // 128)],
                         osem[slot])

    stage_a(jnp.int32(0), 0)

    def pair_body(gp, _):
        for b in range(NBUF):
            g = gp * NBUF + b
            nxt = g + 1

            @pl.when(nxt < nchunk)
            def _():
                stage_a(nxt, (b + 1) % NBUF)

            stage_b(g, b)
        return ()

    lax.fori_loop(0, nchunk // NBUF, pair_body, ())

    for b in range(NBUF):
        g = nchunk - NBUF + b
        base = wbase + g * B
        pltpu.make_async_copy(
            out_v[b], out_hbm.at[pl.ds(base * C // 128, B * C // 128)],
            osem[b]).wait()


_CORE0_FRAC = 0.573


def _make_sc_sample(npad):
    ntot = npad // (NS * B)
    nc0 = max(2, (int(ntot * _CORE0_FRAC) // 2) * 2)
    nc1 = ntot - nc0
    mesh = plsc.VectorSubcoreMesh(
        core_axis_name="c", subcore_axis_name="s",
        num_cores=NC, num_subcores=NS)
    scratch = []
    scratch += [pltpu.VMEM((B,), jnp.float32) for _ in range(2 * NBUF)]
    scratch += [pltpu.VMEM((B,), jnp.int32) for _ in range(4 * NBUF)]
    scratch += [pltpu.VMEM((B,), jnp.float32) for _ in range(4 * NBUF)]
    scratch += [pltpu.VMEM((B, C // 2), jnp.uint32) for _ in range(4 * NBUF)]
    scratch += [pltpu.VMEM((B * C // 128, 128), jnp.float32)
                for _ in range(NBUF)]
    scratch += [pltpu.SemaphoreType.DMA for _ in range(5 * NBUF)]
    return pl.kernel(
        functools.partial(_sc_body, nc0, nc1),
        out_type=jax.ShapeDtypeStruct((npad * C // 128, 128), jnp.float32),
        mesh=mesh,
        compiler_params=pltpu.CompilerParams(
            use_tc_tiling_on_sc=False, needs_layout_passes=False),
        scratch_types=scratch,
    )


def kernel(coords, grid):
    n = coords.shape[0]
    step = NS * B * NBUF
    npad = ((n + step - 1) // step) * step
    table = _transpose(grid, grid, grid, grid).reshape(HW, C // 2)
    ct = coords.T
    if npad != n:
        ct = jnp.pad(ct, ((0, 0), (0, npad - n)))
    out = _make_sc_sample(npad)(ct, table)
    return out.reshape(npad, C)[:n], coords


# speedup vs baseline: 1.2365x; 1.1263x over previous
import functools

import jax
import jax.numpy as jnp
from jax import lax
from jax.experimental import pallas as pl
from jax.experimental.pallas import tpu as pltpu
from jax.experimental.pallas import tpu_sc as plsc

C = 64
SIDE = 1024
HW = SIDE * SIDE
NC, NS, L = 2, 16, 16
NW = NC * NS
B = 128
NBUF = 2


_RB = 8


def _tr_body(ga_ref, gb_ref, t_ref):
    a = ga_ref[...].reshape(C, _RB * SIDE)
    b = gb_ref[...].reshape(C, _RB * SIDE)
    t_ref[:, 0:C] = a.T
    t_ref[:, C:2 * C] = b.T


_transpose = pl.pallas_call(
    _tr_body,
    grid=(SIDE // 2 // _RB,),
    in_specs=[
        pl.BlockSpec((C, _RB, SIDE), lambda i: (0, i, 0)),
        pl.BlockSpec((C, _RB, SIDE), lambda i: (0, i + SIDE // 2 // _RB, 0)),
    ],
    out_specs=pl.BlockSpec((_RB * SIDE, 2 * C), lambda i: (i, 0)),
    out_shape=jax.ShapeDtypeStruct((HW // 2, 2 * C), jnp.float32),
)


def _sc_body(nc0, nc1, coords_hbm, table_hbm, out_hbm, *sc):
    xv = sc[0:2]
    yv = sc[2:4]
    idx = (sc[4:8], sc[8:12])
    wgt = (sc[12:16], sc[16:20])
    taps = (sc[20:24], sc[24:28])
    out_v = sc[28:30]
    gsem = (sc[30:34], sc[34:38])
    osem = sc[38:40]

    cid = lax.axis_index("c")
    sid = lax.axis_index("s")
    nchunk = jnp.where(cid == 0, nc0, nc1)
    wbase = jnp.where(cid == 0, sid * (nc0 * B),
                      NS * nc0 * B + sid * (nc1 * B))

    def stage_a(g, slot):
        base = wbase + g * B
        pltpu.sync_copy(coords_hbm.at[0, pl.ds(base, B)], xv[slot])
        pltpu.sync_copy(coords_hbm.at[1, pl.ds(base, B)], yv[slot])
        for j in range(B // L):
            sj = pl.ds(j * L, L)
            xc = xv[slot][sj]
            yc = yv[slot][sj]
            fx = ((xc + 1.0) * jnp.float32(SIDE) - 1.0) * 0.5
            fy = ((yc + 1.0) * jnp.float32(SIDE) - 1.0) * 0.5
            x0 = fx.astype(jnp.int32)
            y0 = fy.astype(jnp.int32)
            wx1 = fx - x0.astype(jnp.float32)
            wy1 = fy - y0.astype(jnp.float32)
            wx0 = 1.0 - wx1
            wy0 = 1.0 - wy1
            x1 = x0 + 1
            y1 = y0 + 1
            wx1 = jnp.where(x1 <= SIDE - 1, wx1, 0.0)
            wy1 = jnp.where(y1 <= SIDE - 1, wy1, 0.0)
            x1c = jnp.minimum(x1, SIDE - 1)
            y1c = jnp.minimum(y1, SIDE - 1)
            r0 = y0 * SIDE
            r1 = y1c * SIDE
            sl = pl.ds(j * L, L)

            def rmap(p):
                return 2 * (p & (HW // 2 - 1)) + (p >> 19)

            idx[slot][0][sl] = rmap(r0 + x0)
            idx[slot][1][sl] = rmap(r0 + x1c)
            idx[slot][2][sl] = rmap(r1 + x0)
            idx[slot][3][sl] = rmap(r1 + x1c)
            wgt[slot][0][sl] = wx0 * wy0
            wgt[slot][1][sl] = wx1 * wy0
            wgt[slot][2][sl] = wx0 * wy1
            wgt[slot][3][sl] = wx1 * wy1
        for t in range(4):
            pltpu.async_copy(table_hbm.at[idx[slot][t]], taps[slot][t],
                             gsem[slot][t])

    def stage_b(g, slot):
        base = wbase + g * B
        for t in range(4):
            pltpu.make_async_copy(table_hbm.at[idx[slot][t]], taps[slot][t],
                                  gsem[slot][t]).wait()

        @pl.when(g >= NBUF)
        def _():
            prev = wbase + (g - NBUF) * B
            pltpu.make_async_copy(
                out_v[slot],
                out_hbm.at[pl.ds(prev * C // 128, B * C // 128)],
                osem[slot]).wait()

        def blend_body(jg, _):
            sl = pl.ds(jg * L, L)
            wv = [wgt[slot][t][sl] for t in range(4)]
            for lane in range(L):
                i = jg * L + lane
                a = [jnp.full((L,), wv[t][lane], jnp.float32)
                     for t in range(4)]
                orow = jg * (L // 2) + lane // 2
                cb = (lane % 2) * C
                for k in range(C // L):
                    sk = pl.ds(k * L, L)
                    acc = (taps[slot][0][i, sk] * a[0]
                           + taps[slot][1][i, sk] * a[1]
                           + taps[slot][2][i, sk] * a[2]
                           + taps[slot][3][i, sk] * a[3])
                    out_v[slot][orow, pl.ds(cb + k * L, L)] = acc
            return ()

        lax.fori_loop(0, B // L, blend_body, ())
        pltpu.async_copy(out_v[slot],
                         out_hbm.at[pl.ds(base * C // 128, B * C // 128)],
                         osem[slot])

    stage_a(jnp.int32(0), 0)

    def pair_body(gp, _):
        for b in range(NBUF):
            g = gp * NBUF + b
            nxt = g + 1

            @pl.when(nxt < nchunk)
            def _():
                stage_a(nxt, (b + 1) % NBUF)

            stage_b(g, b)
        return ()

    lax.fori_loop(0, nchunk // NBUF, pair_body, ())

    for b in range(NBUF):
        g = nchunk - NBUF + b
        base = wbase + g * B
        pltpu.make_async_copy(
            out_v[b], out_hbm.at[pl.ds(base * C // 128, B * C // 128)],
            osem[b]).wait()


_CORE0_FRAC = 0.557


def _make_sc_sample(npad):
    ntot = npad // (NS * B)
    nc0 = max(2, (int(ntot * _CORE0_FRAC) // 2) * 2)
    nc1 = ntot - nc0
    mesh = plsc.VectorSubcoreMesh(
        core_axis_name="c", subcore_axis_name="s",
        num_cores=NC, num_subcores=NS)
    scratch = []
    scratch += [pltpu.VMEM((B,), jnp.float32) for _ in range(2 * NBUF)]
    scratch += [pltpu.VMEM((B,), jnp.int32) for _ in range(4 * NBUF)]
    scratch += [pltpu.VMEM((B,), jnp.float32) for _ in range(4 * NBUF)]
    scratch += [pltpu.VMEM((B, C), jnp.float32) for _ in range(4 * NBUF)]
    scratch += [pltpu.VMEM((B * C // 128, 128), jnp.float32)
                for _ in range(NBUF)]
    scratch += [pltpu.SemaphoreType.DMA for _ in range(5 * NBUF)]
    return pl.kernel(
        functools.partial(_sc_body, nc0, nc1),
        out_type=jax.ShapeDtypeStruct((npad * C // 128, 128), jnp.float32),
        mesh=mesh,
        compiler_params=pltpu.CompilerParams(
            use_tc_tiling_on_sc=False, needs_layout_passes=False),
        scratch_types=scratch,
    )


def kernel(coords, grid):
    n = coords.shape[0]
    step = NS * B * NBUF
    npad = ((n + step - 1) // step) * step
    table = _transpose(grid, grid).reshape(HW, C)
    ct = coords.T
    if npad != n:
        ct = jnp.pad(ct, ((0, 0), (0, npad - n)))
    out = _make_sc_sample(npad)(ct, table)
    return out.reshape(npad, C)[:n], coords
